# baseline (speedup 1.0000x reference)
import jax
import jax.numpy as jnp
from jax import lax
from jax.experimental import pallas as pl
from jax.experimental.pallas import tpu as pltpu
from jax.experimental.pallas import tpu_sc as plsc

_D = 64
_L = 16
_NC = 2
_NS = 16
_NW = _NC * _NS
_H = 200
_NG = _H // _L
_C = 1
_NB = 4
_MB = 32

_SC_ROWS = 12288
_BLK = 128


def _sc_body(mask_hbm, table_hbm, out_hbm, m0_v, m1_v, tab_v,
             out0_v, out1_v, out2_v, out3_v,
             semm0, semm1, semo0, semo1, semo2, semo3):
    wid = lax.axis_index("s") * _NC + lax.axis_index("c")
    rows_per_w = out_hbm.shape[0] // _NW
    base = wid * rows_per_w
    n_blocks = rows_per_w // _MB

    pltpu.sync_copy(table_hbm, tab_v)
    t0 = [tab_v[0, pl.ds(s * _L, _L)] for s in range(_D // _L)]
    df = [tab_v[1, pl.ds(s * _L, _L)] - t0[s] for s in range(_D // _L)]

    m_bufs = (m0_v, m1_v)
    semms = (semm0, semm1)
    out_bufs = (out0_v, out1_v, out2_v, out3_v)
    semos = (semo0, semo1, semo2, semo3)

    def start_mask(blk, q):
        pltpu.async_copy(
            mask_hbm.at[pl.ds(base + blk * _MB, _MB), :], m_bufs[q], semms[q])

    def wait_mask(q):
        pltpu.make_async_copy(
            mask_hbm.at[pl.ds(base, _MB), :], m_bufs[q], semms[q]).wait()

    def fill_row(m_v, mrow, out_v, c):
        def emit_group(goff, static_h):
            mvec = m_v[mrow, pl.ds(goff, _L)].astype(jnp.float32)
            for u in range(_L):
                mf = mvec[u]
                h = goff + u if static_h is None else static_h + u
                for s in range(_D // _L):
                    out_v[c, h, pl.ds(s * _L, _L)] = t0[s] + mf * df[s]

        def row_group(g, carry):
            emit_group(g * _L, None)
            return carry

        lax.fori_loop(0, _NG, row_group, 0)
        emit_group(_H - _L, _H - _L)

    start_mask(0, 0)

    def block_pair(cc2, carry):
        for q in range(2):
            cc = cc2 * 2 + q
            blk_base = base + cc * _MB
            wait_mask(q)

            @pl.when(cc + 1 < n_blocks)
            def _():
                start_mask(cc + 1, 1 - q)

            def quad_body(jj, carry2):
                for p in range(_NB):
                    j = jj * _NB + p
                    b = cc * (_MB // _C) + j

                    @pl.when(b >= _NB)
                    def _():
                        pltpu.make_async_copy(
                            out_bufs[p], out_hbm.at[pl.ds(base, _C)],
                            semos[p]).wait()

                    for c in range(_C):
                        fill_row(m_bufs[q], j * _C + c, out_bufs[p], c)
                    pltpu.async_copy(
                        out_bufs[p],
                        out_hbm.at[pl.ds(blk_base + j * _C, _C)], semos[p])
                return carry2

            lax.fori_loop(0, _MB // _C // _NB, quad_body, 0)
        return carry

    lax.fori_loop(0, n_blocks // 2, block_pair, 0)
    for p in range(_NB):
        pltpu.make_async_copy(out_bufs[p], out_hbm.at[pl.ds(base, _C)],
                              semos[p]).wait()


def _tc_compute_body(mask_ref, table_ref, out_ref):
    m = mask_ref[...].astype(jnp.float32)
    t = table_ref[...]
    t0 = t[0]
    diff = t[1] - t[0]
    out_ref[...] = t0[None, None, :] + m[:, :, None] * diff[None, None, :]


def _tc_copy_body(buf_ref, _, out_ref):
    out_ref[...] = buf_ref[...]


def kernel(mask, table):
    B, H = mask.shape
    D = table.shape[1]

    sc = pl.kernel(
        _sc_body,
        out_type=jax.ShapeDtypeStruct((_SC_ROWS, H, D), jnp.float32),
        mesh=plsc.VectorSubcoreMesh(core_axis_name="c", subcore_axis_name="s"),
        scratch_types=[
            pltpu.VMEM((_MB, _H), jnp.int32),
            pltpu.VMEM((_MB, _H), jnp.int32),
            pltpu.VMEM((2, _D), jnp.float32),
            pltpu.VMEM((_C, _H, _D), jnp.float32),
            pltpu.VMEM((_C, _H, _D), jnp.float32),
            pltpu.VMEM((_C, _H, _D), jnp.float32),
            pltpu.VMEM((_C, _H, _D), jnp.float32),
            pltpu.SemaphoreType.DMA,
            pltpu.SemaphoreType.DMA,
            pltpu.SemaphoreType.DMA,
            pltpu.SemaphoreType.DMA,
            pltpu.SemaphoreType.DMA,
            pltpu.SemaphoreType.DMA,
        ],
    )
    sc_buf = sc(mask, table)

    tc_rows = B - _SC_ROWS
    blk0 = _SC_ROWS // _BLK
    out0 = pl.pallas_call(
        _tc_compute_body,
        grid=(tc_rows // _BLK,),
        in_specs=[
            pl.BlockSpec((_BLK, H), lambda i: (blk0 + i, 0)),
            pl.BlockSpec((2, D), lambda i: (0, 0)),
        ],
        out_specs=pl.BlockSpec((_BLK, H, D), lambda i: (blk0 + i, 0, 0)),
        out_shape=jax.ShapeDtypeStruct((B, H, D), jnp.float32),
    )(mask, table)

    out = pl.pallas_call(
        _tc_copy_body,
        grid=(_SC_ROWS // _BLK,),
        in_specs=[
            pl.BlockSpec((_BLK, H, D), lambda i: (i, 0, 0)),
            pl.BlockSpec(memory_space=pl.ANY),
        ],
        out_specs=pl.BlockSpec((_BLK, H, D), lambda i: (i, 0, 0)),
        out_shape=jax.ShapeDtypeStruct((B, H, D), jnp.float32),
        input_output_aliases={1: 0},
    )(sc_buf, out0)
    return out


# speedup vs baseline: 1.0760x; 1.0760x over previous
import jax
import jax.numpy as jnp
from jax import lax
from jax.experimental import pallas as pl
from jax.experimental.pallas import tpu as pltpu
from jax.experimental.pallas import tpu_sc as plsc

_D = 64
_L = 16
_NC = 2
_NS = 16
_NW = _NC * _NS
_H = 200
_NG = _H // _L
_C = 1
_NB = 4
_MB = 32

_SC_ROWS = 10240
_BLK = 128


def _sc_body(mask_hbm, table_hbm, out_hbm, m0_v, m1_v, tab_v,
             out0_v, out1_v, out2_v, out3_v,
             semm0, semm1, semo0, semo1, semo2, semo3):
    wid = lax.axis_index("s") * _NC + lax.axis_index("c")
    rows_per_w = out_hbm.shape[0] // _NW
    base = wid * rows_per_w
    n_blocks = rows_per_w // _MB

    pltpu.sync_copy(table_hbm, tab_v)
    t0 = [tab_v[0, pl.ds(s * _L, _L)] for s in range(_D // _L)]
    df = [tab_v[1, pl.ds(s * _L, _L)] - t0[s] for s in range(_D // _L)]

    m_bufs = (m0_v, m1_v)
    semms = (semm0, semm1)
    out_bufs = (out0_v, out1_v, out2_v, out3_v)
    semos = (semo0, semo1, semo2, semo3)

    def start_mask(blk, q):
        pltpu.async_copy(
            mask_hbm.at[pl.ds(base + blk * _MB, _MB), :], m_bufs[q], semms[q])

    def wait_mask(q):
        pltpu.make_async_copy(
            mask_hbm.at[pl.ds(base, _MB), :], m_bufs[q], semms[q]).wait()

    def fill_row(m_v, mrow, out_v, c):
        def emit_group(goff, static_h):
            mvec = m_v[mrow, pl.ds(goff, _L)].astype(jnp.float32)
            for u in range(_L):
                mf = mvec[u]
                h = goff + u if static_h is None else static_h + u
                for s in range(_D // _L):
                    out_v[c, h, pl.ds(s * _L, _L)] = t0[s] + mf * df[s]

        def row_group(g, carry):
            emit_group(g * _L, None)
            return carry

        lax.fori_loop(0, _NG, row_group, 0)
        emit_group(_H - _L, _H - _L)

    start_mask(0, 0)

    def block_pair(cc2, carry):
        for q in range(2):
            cc = cc2 * 2 + q
            blk_base = base + cc * _MB
            wait_mask(q)

            @pl.when(cc + 1 < n_blocks)
            def _():
                start_mask(cc + 1, 1 - q)

            def quad_body(jj, carry2):
                for p in range(_NB):
                    j = jj * _NB + p
                    b = cc * (_MB // _C) + j

                    @pl.when(b >= _NB)
                    def _():
                        pltpu.make_async_copy(
                            out_bufs[p], out_hbm.at[pl.ds(base, _C)],
                            semos[p]).wait()

                    for c in range(_C):
                        fill_row(m_bufs[q], j * _C + c, out_bufs[p], c)
                    pltpu.async_copy(
                        out_bufs[p],
                        out_hbm.at[pl.ds(blk_base + j * _C, _C)], semos[p])
                return carry2

            lax.fori_loop(0, _MB // _C // _NB, quad_body, 0)
        return carry

    lax.fori_loop(0, n_blocks // 2, block_pair, 0)
    for p in range(_NB):
        pltpu.make_async_copy(out_bufs[p], out_hbm.at[pl.ds(base, _C)],
                              semos[p]).wait()


def _tc_compute_body(mask_ref, table_ref, out_ref):
    m = mask_ref[...].astype(jnp.float32)
    t = table_ref[...]
    t0 = t[0]
    diff = t[1] - t[0]
    out_ref[...] = t0[None, None, :] + m[:, :, None] * diff[None, None, :]


def _tc_copy_body(buf_ref, _, out_ref):
    out_ref[...] = buf_ref[...]


def kernel(mask, table):
    B, H = mask.shape
    D = table.shape[1]

    sc = pl.kernel(
        _sc_body,
        out_type=jax.ShapeDtypeStruct((_SC_ROWS, H, D), jnp.float32),
        mesh=plsc.VectorSubcoreMesh(core_axis_name="c", subcore_axis_name="s"),
        scratch_types=[
            pltpu.VMEM((_MB, _H), jnp.int32),
            pltpu.VMEM((_MB, _H), jnp.int32),
            pltpu.VMEM((2, _D), jnp.float32),
            pltpu.VMEM((_C, _H, _D), jnp.float32),
            pltpu.VMEM((_C, _H, _D), jnp.float32),
            pltpu.VMEM((_C, _H, _D), jnp.float32),
            pltpu.VMEM((_C, _H, _D), jnp.float32),
            pltpu.SemaphoreType.DMA,
            pltpu.SemaphoreType.DMA,
            pltpu.SemaphoreType.DMA,
            pltpu.SemaphoreType.DMA,
            pltpu.SemaphoreType.DMA,
            pltpu.SemaphoreType.DMA,
        ],
    )
    sc_buf = sc(mask, table)

    tc_rows = B - _SC_ROWS
    blk0 = _SC_ROWS // _BLK
    tc_buf = pl.pallas_call(
        _tc_compute_body,
        grid=(tc_rows // _BLK,),
        in_specs=[
            pl.BlockSpec((_BLK, H), lambda i: (blk0 + i, 0)),
            pl.BlockSpec((2, D), lambda i: (0, 0)),
        ],
        out_specs=pl.BlockSpec((_BLK, H, D), lambda i: (i, 0, 0)),
        out_shape=jax.ShapeDtypeStruct((tc_rows, H, D), jnp.float32),
    )(mask, table)

    return jnp.concatenate([sc_buf, tc_buf], axis=0)


# speedup vs baseline: 1.4435x; 1.3415x over previous
import jax
import jax.numpy as jnp
from jax import lax
from jax.experimental import pallas as pl
from jax.experimental.pallas import tpu as pltpu
from jax.experimental.pallas import tpu_sc as plsc

_D = 64
_L = 16
_NC = 2
_NS = 16
_NW = _NC * _NS
_H = 200
_NG = _H // _L
_C = 2
_MB = 32


def _sc_body(mask_hbm, table_hbm, out_hbm, m0_v, m1_v, tab_v, out0_v, out1_v,
             semm0, semm1, semo0, semo1):
    wid = lax.axis_index("s") * _NC + lax.axis_index("c")
    rows_per_w = out_hbm.shape[0] // _NW
    base = wid * rows_per_w
    n_blocks = rows_per_w // _MB

    pltpu.sync_copy(table_hbm, tab_v)
    t0 = [tab_v[0, pl.ds(s * _L, _L)] for s in range(_D // _L)]
    df = [tab_v[1, pl.ds(s * _L, _L)] - t0[s] for s in range(_D // _L)]

    m_bufs = (m0_v, m1_v)
    semms = (semm0, semm1)
    out_bufs = (out0_v, out1_v)
    semos = (semo0, semo1)

    def start_mask(blk, q):
        pltpu.async_copy(
            mask_hbm.at[pl.ds(base + blk * _MB, _MB), :], m_bufs[q], semms[q])

    def wait_mask(q):
        pltpu.make_async_copy(
            mask_hbm.at[pl.ds(base, _MB), :], m_bufs[q], semms[q]).wait()

    def fill_row(m_v, mrow, out_v, c):
        def emit_group(goff, static_h):
            mvec = m_v[mrow, pl.ds(goff, _L)].astype(jnp.float32)
            for u in range(_L):
                mf = mvec[u]
                h = goff + u if static_h is None else static_h + u
                for s in range(_D // _L):
                    out_v[c, h, pl.ds(s * _L, _L)] = t0[s] + mf * df[s]

        def row_group(g, carry):
            emit_group(g * _L, None)
            return carry

        lax.fori_loop(0, _NG, row_group, 0)
        emit_group(_H - _L, _H - _L)

    start_mask(0, 0)

    def block_pair(cc2, carry):
        for q in range(2):
            cc = cc2 * 2 + q
            blk_base = base + cc * _MB
            wait_mask(q)

            @pl.when(cc + 1 < n_blocks)
            def _():
                start_mask(cc + 1, 1 - q)

            def pair_body(jj, carry2):
                for p in range(2):
                    j = jj * 2 + p
                    b = cc * (_MB // _C) + j

                    @pl.when(b >= 2)
                    def _():
                        pltpu.make_async_copy(
                            out_bufs[p], out_hbm.at[pl.ds(base, _C)],
                            semos[p]).wait()

                    for c in range(_C):
                        fill_row(m_bufs[q], j * _C + c, out_bufs[p], c)
                    pltpu.async_copy(
                        out_bufs[p],
                        out_hbm.at[pl.ds(blk_base + j * _C, _C)], semos[p])
                return carry2

            lax.fori_loop(0, _MB // _C // 2, pair_body, 0)
        return carry

    lax.fori_loop(0, n_blocks // 2, block_pair, 0)
    for p in range(2):
        pltpu.make_async_copy(out_bufs[p], out_hbm.at[pl.ds(base, _C)],
                              semos[p]).wait()


def kernel(mask, table):
    B, H = mask.shape
    D = table.shape[1]

    k = pl.kernel(
        _sc_body,
        out_type=jax.ShapeDtypeStruct((B, H, D), jnp.float32),
        mesh=plsc.VectorSubcoreMesh(core_axis_name="c", subcore_axis_name="s"),
        scratch_types=[
            pltpu.VMEM((_MB, _H), jnp.int32),
            pltpu.VMEM((_MB, _H), jnp.int32),
            pltpu.VMEM((2, _D), jnp.float32),
            pltpu.VMEM((_C, _H, _D), jnp.float32),
            pltpu.VMEM((_C, _H, _D), jnp.float32),
            pltpu.SemaphoreType.DMA,
            pltpu.SemaphoreType.DMA,
            pltpu.SemaphoreType.DMA,
            pltpu.SemaphoreType.DMA,
        ],
    )
    return k(mask, table)
